# Initial kernel scaffold; baseline (speedup 1.0000x reference)
import functools

import jax
import jax.numpy as jnp
from jax import lax
from jax.experimental import pallas as pl
from jax.experimental.pallas import tpu as pltpu
from jax.experimental.pallas import tpu_sc as plsc

NUM_EMB = 1_000_000
VEC = 128
RED = 32
BATCH = 4096
SEQ = 200


ROW_BLOCK = 8000


def _proj_body(t_ref, w_ref, b_ref, o_ref):
    o_ref[...] = lax.dot_general(
        t_ref[...], w_ref[...],
        dimension_numbers=(((1,), (1,)), ((), ())),
        preferred_element_type=jnp.float32,
    ) + b_ref[...]


def _project(table, W, b):
    return pl.pallas_call(
        _proj_body,
        grid=(NUM_EMB // ROW_BLOCK,),
        in_specs=[
            pl.BlockSpec((ROW_BLOCK, VEC), lambda i: (i, 0)),
            pl.BlockSpec((RED, VEC), lambda i: (0, 0)),
            pl.BlockSpec((1, RED), lambda i: (0, 0)),
        ],
        out_specs=pl.BlockSpec((ROW_BLOCK, RED), lambda i: (i, 0)),
        out_shape=jax.ShapeDtypeStruct((NUM_EMB, RED), jnp.float32),
    )(table, W, b.reshape(1, RED))


_B = BATCH * SEQ
_NW = 32
_BPW = _B // _NW
_G = 128
_KG = 8
_C = _G * _KG
_NCHUNK = _BPW // _C


def _gather_body(idx_hbm, proj_hbm, out_hbm, idx_v, rows_v, sem):
    cid = lax.axis_index("c")
    sid = lax.axis_index("s")
    wid = sid * 2 + cid
    base = wid * _BPW

    def chunk(ci, carry):
        off = base + ci * _C
        pltpu.sync_copy(idx_hbm.at[pl.ds(off // _G, _KG)], idx_v)
        copies = []
        for j in range(_KG):
            copies.append(
                pltpu.async_copy(
                    proj_hbm.at[idx_v.at[j]],
                    rows_v.at[pl.ds(j * _G, _G)],
                    sem,
                )
            )
        for cp in copies:
            cp.wait()
        pltpu.sync_copy(rows_v, out_hbm.at[pl.ds(off, _C)])
        return carry

    lax.fori_loop(0, _NCHUNK, chunk, 0)


@functools.partial(
    pl.kernel,
    mesh=plsc.VectorSubcoreMesh(core_axis_name="c", subcore_axis_name="s"),
    out_type=jax.ShapeDtypeStruct((_B, RED), jnp.float32),
    scratch_types=[
        pltpu.VMEM((_KG, _G), jnp.int32),
        pltpu.VMEM((_C, RED), jnp.float32),
        pltpu.SemaphoreType.DMA,
    ],
)
def _gather(idx_hbm, proj_hbm, out_hbm, idx_v, rows_v, sem):
    _gather_body(idx_hbm, proj_hbm, out_hbm, idx_v, rows_v, sem)


def kernel(x, table, W, b):
    proj = _project(table, W, b)
    idx = x.reshape(_B // _G, _G).astype(jnp.int32)
    out = _gather(idx, proj)
    return out.reshape(BATCH, SEQ, RED)


# speedup vs baseline: 12.2067x; 12.2067x over previous
import functools

import jax
import jax.numpy as jnp
from jax import lax
from jax.experimental import pallas as pl
from jax.experimental.pallas import tpu as pltpu
from jax.experimental.pallas import tpu_sc as plsc

NUM_EMB = 1_000_000
VEC = 128
RED = 32
BATCH = 4096
SEQ = 200


ROW_BLOCK = 8000


def _proj_body(t_ref, w_ref, b_ref, o_ref):
    o_ref[...] = lax.dot_general(
        t_ref[...], w_ref[...],
        dimension_numbers=(((1,), (1,)), ((), ())),
        preferred_element_type=jnp.float32,
    ) + b_ref[...]


def _project(table, W, b):
    return pl.pallas_call(
        _proj_body,
        grid=(NUM_EMB // ROW_BLOCK,),
        in_specs=[
            pl.BlockSpec((ROW_BLOCK, VEC), lambda i: (i, 0)),
            pl.BlockSpec((RED, VEC), lambda i: (0, 0)),
            pl.BlockSpec((1, RED), lambda i: (0, 0)),
        ],
        out_specs=pl.BlockSpec((ROW_BLOCK, RED), lambda i: (i, 0)),
        out_shape=jax.ShapeDtypeStruct((NUM_EMB, RED), jnp.float32),
    )(table, W, b.reshape(1, RED))


_B = BATCH * SEQ
_NW = 32
_BPW = _B // _NW
_G = 128
_KG = 8
_C = _G * _KG
_NCHUNK = _BPW // _C


def _gather_body(idx_hbm, proj_hbm, out_hbm, idx_v, rows_v, sem):
    cid = lax.axis_index("c")
    sid = lax.axis_index("s")
    wid = sid * 2 + cid
    base = wid * _BPW

    def chunk(ci, carry):
        off = pl.multiple_of(base + ci * _C, _C)
        row = pl.multiple_of(off // _G, _KG)
        pltpu.sync_copy(idx_hbm.at[pl.ds(row, _KG)], idx_v)
        copies = []
        for j in range(_KG):
            copies.append(
                pltpu.async_copy(
                    proj_hbm.at[idx_v.at[j]],
                    rows_v.at[pl.ds(j * _G, _G)],
                    sem,
                )
            )
        for cp in copies:
            cp.wait()
        pltpu.sync_copy(rows_v, out_hbm.at[pl.ds(off, _C)])
        return carry

    lax.fori_loop(0, _NCHUNK, chunk, 0)


@functools.partial(
    pl.kernel,
    mesh=plsc.VectorSubcoreMesh(core_axis_name="c", subcore_axis_name="s"),
    compiler_params=pltpu.CompilerParams(use_tc_tiling_on_sc=False),
    out_type=jax.ShapeDtypeStruct((_B, RED), jnp.float32),
    scratch_types=[
        pltpu.VMEM((_KG, _G), jnp.int32),
        pltpu.VMEM((_C, RED), jnp.float32),
        pltpu.SemaphoreType.DMA,
    ],
)
def _gather(idx_hbm, proj_hbm, out_hbm, idx_v, rows_v, sem):
    _gather_body(idx_hbm, proj_hbm, out_hbm, idx_v, rows_v, sem)


def kernel(x, table, W, b):
    proj = _project(table, W, b)
    idx = x.reshape(_B // _G, _G).astype(jnp.int32)
    out = _gather(idx, proj)
    return out.reshape(BATCH, SEQ, RED)


# speedup vs baseline: 27.2467x; 2.2321x over previous
import functools

import jax
import jax.numpy as jnp
from jax import lax
from jax.experimental import pallas as pl
from jax.experimental.pallas import tpu as pltpu
from jax.experimental.pallas import tpu_sc as plsc

NUM_EMB = 1_000_000
VEC = 128
RED = 32
BATCH = 4096
SEQ = 200


PACK = 4
ROWS_P = NUM_EMB // PACK
QBLK = 2000
NBLK = ROWS_P // QBLK


def _proj_body(t0, t1, t2, t3, w_ref, b_ref, o_ref):
    parts = []
    for t in (t0, t1, t2, t3):
        parts.append(
            lax.dot_general(
                t[...], w_ref[...],
                dimension_numbers=(((1,), (1,)), ((), ())),
                preferred_element_type=jnp.float32,
            )
        )
    o_ref[...] = jnp.concatenate(parts, axis=1) + b_ref[...]


def _project(table, W, b):
    b4 = jnp.tile(b, PACK).reshape(1, PACK * RED)
    t_spec = lambda q: pl.BlockSpec((QBLK, VEC), lambda i, q=q: (q * NBLK + i, 0))
    return pl.pallas_call(
        _proj_body,
        grid=(NBLK,),
        in_specs=[
            t_spec(0), t_spec(1), t_spec(2), t_spec(3),
            pl.BlockSpec((RED, VEC), lambda i: (0, 0)),
            pl.BlockSpec((1, PACK * RED), lambda i: (0, 0)),
        ],
        out_specs=pl.BlockSpec((QBLK, PACK * RED), lambda i: (i, 0)),
        out_shape=jax.ShapeDtypeStruct((ROWS_P, PACK * RED), jnp.float32),
    )(table, table, table, table, W, b4)


_B = BATCH * SEQ
_NW = 32
_BPW = _B // _NW
_G = 128
_KG = 8
_C = _G * _KG
_NCHUNK = _BPW // _C
_L = 16


def _remap_chunk(idx_v):
    for j in range(_KG):
        for k in range(_G // _L):
            g = idx_v[j, pl.ds(k * _L, _L)]
            q = (
                jnp.where(g >= ROWS_P, 1, 0)
                + jnp.where(g >= 2 * ROWS_P, 1, 0)
                + jnp.where(g >= 3 * ROWS_P, 1, 0)
            ).astype(jnp.int32)
            idx_v[j, pl.ds(k * _L, _L)] = (g << 2) - q * (NUM_EMB - 1)


def _gather_body(idx_hbm, proj_hbm, out_hbm, idx_v, rows_v, sem):
    cid = lax.axis_index("c")
    sid = lax.axis_index("s")
    wid = sid * 2 + cid
    base = wid * _BPW

    def chunk(ci, carry):
        off = pl.multiple_of(base + ci * _C, _C)
        row = pl.multiple_of(off // _G, _KG)
        pltpu.sync_copy(idx_hbm.at[pl.ds(row, _KG)], idx_v)
        _remap_chunk(idx_v)
        copies = []
        for j in range(_KG):
            copies.append(
                pltpu.async_copy(
                    proj_hbm.at[idx_v.at[j]],
                    rows_v.at[pl.ds(j * _G, _G)],
                    sem,
                )
            )
        for cp in copies:
            cp.wait()
        pltpu.sync_copy(rows_v, out_hbm.at[pl.ds(off, _C)])
        return carry

    lax.fori_loop(0, _NCHUNK, chunk, 0)


@functools.cache
def _gather_kernel():
    return pl.kernel(
        _gather_body,
        mesh=plsc.VectorSubcoreMesh(core_axis_name="c", subcore_axis_name="s"),
        compiler_params=pltpu.CompilerParams(use_tc_tiling_on_sc=False),
        out_type=jax.ShapeDtypeStruct((_B, RED), jnp.float32),
        scratch_types=[
            pltpu.VMEM((_KG, _G), jnp.int32),
            pltpu.VMEM((_C, RED), jnp.float32),
            pltpu.SemaphoreType.DMA,
        ],
    )


_BT = BATCH // VEC
_SR = SEQ * RED
_PB = _B // PACK


def _tr_body(t_ref, o_ref):
    x3 = t_ref[0].reshape(VEC, _SR // VEC, VEC)
    o_ref[...] = x3.transpose(1, 2, 0).reshape(_SR, VEC)


def _transpose(out_flat):
    x = out_flat.reshape(_BT, _PB // _BT, VEC)
    return pl.pallas_call(
        _tr_body,
        grid=(_BT,),
        in_specs=[pl.BlockSpec((1, _PB // _BT, VEC), lambda i: (i, 0, 0))],
        out_specs=pl.BlockSpec((_SR, VEC), lambda i: (0, i)),
        out_shape=jax.ShapeDtypeStruct((_SR, BATCH), jnp.float32),
    )(x)


def kernel(x, table, W, b):
    proj = _project(table, W, b).reshape(NUM_EMB, RED)
    idx = x.reshape(_B // _G, _G).astype(jnp.int32)
    out_flat = _gather_kernel()(idx, proj)
    out2 = _transpose(out_flat.reshape(_PB, VEC))
    return out2.reshape(SEQ, RED, BATCH).transpose(2, 0, 1)


# speedup vs baseline: 32.1804x; 1.1811x over previous
import functools

import jax
import jax.numpy as jnp
from jax import lax
from jax.experimental import pallas as pl
from jax.experimental.pallas import tpu as pltpu
from jax.experimental.pallas import tpu_sc as plsc

NUM_EMB = 1_000_000
VEC = 128
RED = 32
BATCH = 4096
SEQ = 200


PACK = 4
ROWS_P = NUM_EMB // PACK
QBLK = 2000
NBLK = ROWS_P // QBLK


def _proj_body(t0, t1, t2, t3, w_ref, b_ref, o_ref):
    parts = []
    for t in (t0, t1, t2, t3):
        parts.append(
            lax.dot_general(
                t[...], w_ref[...],
                dimension_numbers=(((1,), (1,)), ((), ())),
                preferred_element_type=jnp.float32,
            )
        )
    o_ref[...] = jnp.concatenate(parts, axis=1) + b_ref[...]


def _project(table, W, b):
    b4 = jnp.tile(b, PACK).reshape(1, PACK * RED)
    t_spec = lambda q: pl.BlockSpec((QBLK, VEC), lambda i, q=q: (q * NBLK + i, 0))
    return pl.pallas_call(
        _proj_body,
        grid=(NBLK,),
        in_specs=[
            t_spec(0), t_spec(1), t_spec(2), t_spec(3),
            pl.BlockSpec((RED, VEC), lambda i: (0, 0)),
            pl.BlockSpec((1, PACK * RED), lambda i: (0, 0)),
        ],
        out_specs=pl.BlockSpec((QBLK, PACK * RED), lambda i: (i, 0)),
        out_shape=jax.ShapeDtypeStruct((ROWS_P, PACK * RED), jnp.float32),
    )(table, table, table, table, W, b4)


_B = BATCH * SEQ
_NW = 32
_BPW = _B // _NW
_G = 128
_KG = 8
_C = _G * _KG
_NCHUNK = _BPW // _C
_L = 16


def _remap_chunk(idx_v):
    for j in range(_KG):
        for k in range(_G // _L):
            g = idx_v[j, pl.ds(k * _L, _L)]
            q = (
                jnp.where(g >= ROWS_P, 1, 0)
                + jnp.where(g >= 2 * ROWS_P, 1, 0)
                + jnp.where(g >= 3 * ROWS_P, 1, 0)
            ).astype(jnp.int32)
            idx_v[j, pl.ds(k * _L, _L)] = (g << 2) - q * (NUM_EMB - 1)


def _dst_chunk(dst_v, base, n0):
    iota = lax.iota(jnp.int32, _L)
    for j in range(_KG):
        for k in range(_G // _L):
            n = n0 + (j * _G + k * _L) + iota
            bl = (n * 5243) >> 20
            s = n - bl * 200
            si = s >> 2
            sl = s - (si << 2)
            dst_v[j, pl.ds(k * _L, _L)] = base + (si << 9) + (bl << 2) + sl


def _gather_body(idx_hbm, proj_hbm, out_hbm, idx_v, dst_v, rows_v, sem):
    cid = lax.axis_index("c")
    sid = lax.axis_index("s")
    wid = sid * 2 + cid
    base = wid * _BPW

    def chunk(ci, carry):
        off = pl.multiple_of(base + ci * _C, _C)
        row = pl.multiple_of(off // _G, _KG)
        pltpu.sync_copy(idx_hbm.at[pl.ds(row, _KG)], idx_v)
        _remap_chunk(idx_v)
        _dst_chunk(dst_v, base, ci * _C)
        copies = []
        for j in range(_KG):
            copies.append(
                pltpu.async_copy(
                    proj_hbm.at[idx_v.at[j]],
                    rows_v.at[pl.ds(j * _G, _G)],
                    sem,
                )
            )
        for cp in copies:
            cp.wait()
        copies = []
        for j in range(_KG):
            copies.append(
                pltpu.async_copy(
                    rows_v.at[pl.ds(j * _G, _G)],
                    out_hbm.at[dst_v.at[j]],
                    sem,
                )
            )
        for cp in copies:
            cp.wait()
        return carry

    lax.fori_loop(0, _NCHUNK, chunk, 0)


@functools.cache
def _gather_kernel():
    return pl.kernel(
        _gather_body,
        mesh=plsc.VectorSubcoreMesh(core_axis_name="c", subcore_axis_name="s"),
        compiler_params=pltpu.CompilerParams(use_tc_tiling_on_sc=False),
        out_type=jax.ShapeDtypeStruct((_B, RED), jnp.float32),
        scratch_types=[
            pltpu.VMEM((_KG, _G), jnp.int32),
            pltpu.VMEM((_KG, _G), jnp.int32),
            pltpu.VMEM((_C, RED), jnp.float32),
            pltpu.SemaphoreType.DMA,
        ],
    )


_BT = BATCH // VEC
_SR = SEQ * RED
_PB = _B // PACK


def _tr_body(t_ref, o_ref):
    x3 = t_ref[0].reshape(_SR // VEC, VEC, VEC)
    o_ref[...] = x3.transpose(0, 2, 1).reshape(_SR, VEC)


def _transpose(out_flat):
    x = out_flat.reshape(_BT, _PB // _BT, VEC)
    return pl.pallas_call(
        _tr_body,
        grid=(_BT,),
        in_specs=[pl.BlockSpec((1, _PB // _BT, VEC), lambda i: (i, 0, 0))],
        out_specs=pl.BlockSpec((_SR, VEC), lambda i: (0, i)),
        out_shape=jax.ShapeDtypeStruct((_SR, BATCH), jnp.float32),
    )(x)


def kernel(x, table, W, b):
    proj = _project(table, W, b).reshape(NUM_EMB, RED)
    idx = x.reshape(_B // _G, _G).astype(jnp.int32)
    out_flat = _gather_kernel()(idx, proj)
    out2 = _transpose(out_flat.reshape(_PB, VEC))
    return out2.reshape(SEQ, RED, BATCH).transpose(2, 0, 1)


# speedup vs baseline: 35.0582x; 1.0894x over previous
import functools

import jax
import jax.numpy as jnp
from jax import lax
from jax.experimental import pallas as pl
from jax.experimental.pallas import tpu as pltpu
from jax.experimental.pallas import tpu_sc as plsc

NUM_EMB = 1_000_000
VEC = 128
RED = 32
BATCH = 4096
SEQ = 200


PACK = 4
ROWS_P = NUM_EMB // PACK
QBLK = 2000
NBLK = ROWS_P // QBLK


def _proj_body(t0, t1, t2, t3, w_ref, b_ref, o_ref):
    parts = []
    for t in (t0, t1, t2, t3):
        parts.append(
            lax.dot_general(
                t[...], w_ref[...],
                dimension_numbers=(((1,), (1,)), ((), ())),
                preferred_element_type=jnp.float32,
            )
        )
    o_ref[...] = jnp.concatenate(parts, axis=1) + b_ref[...]


def _project(table, W, b):
    b4 = jnp.tile(b, PACK).reshape(1, PACK * RED)
    t_spec = lambda q: pl.BlockSpec((QBLK, VEC), lambda i, q=q: (q * NBLK + i, 0))
    return pl.pallas_call(
        _proj_body,
        grid=(NBLK,),
        in_specs=[
            t_spec(0), t_spec(1), t_spec(2), t_spec(3),
            pl.BlockSpec((RED, VEC), lambda i: (0, 0)),
            pl.BlockSpec((1, PACK * RED), lambda i: (0, 0)),
        ],
        out_specs=pl.BlockSpec((QBLK, PACK * RED), lambda i: (i, 0)),
        out_shape=jax.ShapeDtypeStruct((ROWS_P, PACK * RED), jnp.float32),
    )(table, table, table, table, W, b4)


_B = BATCH * SEQ
_NW = 32
_BPW = _B // _NW
_G = 128
_KG = 8
_C = _G * _KG
_NCHUNK = _BPW // _C
_L = 16


_NST = SEQ // _KG


def _remap_slab(slab_v):
    def row(st, carry):
        for k in range(_KG * _G // _L):
            g = slab_v[st, pl.ds(k * _L, _L)]
            q = (
                jnp.where(g >= ROWS_P, 1, 0)
                + jnp.where(g >= 2 * ROWS_P, 1, 0)
                + jnp.where(g >= 3 * ROWS_P, 1, 0)
            ).astype(jnp.int32)
            slab_v[st, pl.ds(k * _L, _L)] = (g << 2) - q * (NUM_EMB - 1)
        return carry

    lax.fori_loop(0, _NST, row, 0)


def _gather_body(xq_hbm, proj_hbm, out_hbm, slab_v, dst_v, rows_v, gsem, ssem):
    cid = lax.axis_index("c")
    sid = lax.axis_index("s")
    wid = sid * 2 + cid
    base = wid * _BPW

    col = pl.multiple_of(wid * _C, _C)
    pltpu.sync_copy(xq_hbm.at[pl.ds(0, _NST), pl.ds(col, _C)], slab_v)
    _remap_slab(slab_v)

    iota = lax.iota(jnp.int32, _L)

    def fire_g(ci, buf):
        return [
            pltpu.async_copy(
                proj_hbm.at[slab_v.at[ci, pl.ds(j * _G, _G)]],
                rows_v.at[buf, pl.ds(j * _G, _G)],
                gsem,
            )
            for j in range(_KG)
        ]

    def fire_s(ci, buf):
        for j in range(_KG):
            for k in range(_G // _L):
                bl = k * _L + iota
                s = ci * _KG + j
                si = s >> 2
                dst_v[buf, j, pl.ds(k * _L, _L)] = (
                    base + (si << 9) + (bl << 2) + (s - (si << 2))
                )
        return [
            pltpu.async_copy(
                rows_v.at[buf, pl.ds(j * _G, _G)],
                out_hbm.at[dst_v.at[buf, j]],
                ssem,
            )
            for j in range(_KG)
        ]

    def drain(copies):
        for cp in copies:
            cp.wait()

    drain(fire_g(0, 0))

    def pair(st2, carry):
        c0 = st2 * 2
        ss0 = fire_s(c0, 0)
        gs1 = fire_g(c0 + 1, 1)
        drain(gs1)
        drain(ss0)
        ss1 = fire_s(c0 + 1, 1)
        gs2 = fire_g(c0 + 2, 0)
        drain(gs2)
        drain(ss1)
        return carry

    lax.fori_loop(0, _NST // 2, pair, 0)
    drain(fire_s(_NST - 1, 0))


@functools.cache
def _gather_kernel():
    return pl.kernel(
        _gather_body,
        mesh=plsc.VectorSubcoreMesh(core_axis_name="c", subcore_axis_name="s"),
        compiler_params=pltpu.CompilerParams(use_tc_tiling_on_sc=False),
        out_type=jax.ShapeDtypeStruct((_B, RED), jnp.float32),
        scratch_types=[
            pltpu.VMEM((_NST, _C), jnp.int32),
            pltpu.VMEM((2, _KG, _G), jnp.int32),
            pltpu.VMEM((2, _C, RED), jnp.float32),
            pltpu.SemaphoreType.DMA,
            pltpu.SemaphoreType.DMA,
        ],
    )


_BT = BATCH // VEC
_SR = SEQ * RED
_PB = _B // PACK


def _tr_body(t_ref, o_ref):
    x3 = t_ref[0].reshape(_SR // VEC, VEC, VEC)
    o_ref[...] = x3.transpose(0, 2, 1).reshape(_SR, VEC)


def _transpose(out_flat):
    x = out_flat.reshape(_BT, _PB // _BT, VEC)
    return pl.pallas_call(
        _tr_body,
        grid=(_BT,),
        in_specs=[pl.BlockSpec((1, _PB // _BT, VEC), lambda i: (i, 0, 0))],
        out_specs=pl.BlockSpec((_SR, VEC), lambda i: (0, i)),
        out_shape=jax.ShapeDtypeStruct((_SR, BATCH), jnp.float32),
    )(x)


def kernel(x, table, W, b):
    proj = _project(table, W, b).reshape(NUM_EMB, RED)
    xq = (
        x.astype(jnp.int32)
        .transpose(1, 0)
        .reshape(_NST, _KG, BATCH // VEC, VEC)
        .transpose(0, 2, 1, 3)
        .reshape(_NST, BATCH // VEC * _KG * VEC)
    )
    out_flat = _gather_kernel()(xq, proj)
    out2 = _transpose(out_flat.reshape(_PB, VEC))
    return out2.reshape(SEQ, RED, BATCH).transpose(2, 0, 1)
